# baseline (speedup 1.0000x reference)
import functools

import jax
import jax.numpy as jnp
from jax.experimental import pallas as pl
from jax.experimental.pallas import tpu as pltpu
from jax.experimental.pallas import tpu_sc as plsc

T = 2048
H = 1024
I = 512
E = 8
ROUTED_SCALE = 2.5

BR = 128
NTILES = T * 2 // BR + E
CAP = NTILES * BR
BT = 256

NW = 32
TPW = T // NW


HP = H // 2


def _pack_bf16_pair(lo_f32, hi_f32):
    def bits(v):
        r = jax.lax.bitcast_convert_type(v, jnp.int32)
        lsb = jax.lax.shift_right_logical(r, 16) & 1
        return jax.lax.shift_right_logical(r + 0x7FFF + lsb, 16)
    return jax.lax.shift_left(bits(hi_f32), 16) | bits(lo_f32)


def _unpack_bf16_pair(pk):
    lo = jax.lax.bitcast_convert_type(jax.lax.shift_left(pk, 16), jnp.float32)
    hi = jax.lax.bitcast_convert_type(
        jax.lax.shift_left(jax.lax.shift_right_arithmetic(pk, 16), 16),
        jnp.float32)
    return lo, hi


def _router_body(x_ref, gate_ref, pos_ref, wts_ref, te_ref, xpk_ref):
    lane8 = jax.lax.broadcasted_iota(jnp.int32, (T, E), 1)
    xb = x_ref[...].astype(jnp.bfloat16)
    logits = jax.lax.dot_general(
        xb, gate_ref[...].astype(jnp.bfloat16), (((1,), (1,)), ((), ())),
        preferred_element_type=jnp.float32)
    m = jnp.max(logits, axis=1, keepdims=True)
    ex = jnp.exp(logits - m)
    scores = ex / jnp.sum(ex, axis=1, keepdims=True)
    neg = jnp.float32(-1e30)
    m0 = jnp.max(jnp.where(lane8 < 4, scores, neg), axis=1, keepdims=True)
    m1 = jnp.max(jnp.where(lane8 >= 4, scores, neg), axis=1, keepdims=True)
    low_f = jnp.where(lane8 < 4, 1.0, 0.0)
    pick_low = jnp.where(m0 >= m1, 1.0, 0.0)
    gmask = pick_low * low_f + (1.0 - pick_low) * (1.0 - low_f)
    ms = scores * gmask
    v1 = jnp.max(ms, axis=1, keepdims=True)
    i1 = jnp.min(jnp.where(ms == v1, lane8, E), axis=1, keepdims=True)
    ms2 = jnp.where(lane8 == i1, -1.0, ms)
    v2 = jnp.max(ms2, axis=1, keepdims=True)
    i2 = jnp.min(jnp.where(ms2 == v2, lane8, E), axis=1, keepdims=True)
    denom = v1 + v2 + 1e-20
    wts_ref[:, 0:1] = v1 / denom * ROUTED_SCALE
    wts_ref[:, 1:2] = v2 / denom * ROUTED_SCALE

    oh = (jnp.where(lane8 == i1, 1.0, 0.0)
          + jnp.where(lane8 == i2, 1.0, 0.0))
    cum = oh
    s = 1
    while s < T:
        cum = cum + jnp.concatenate(
            [jnp.zeros((s, E), jnp.float32), cum[:T - s, :]], axis=0)
        s *= 2
    rank = cum - oh
    counts = jnp.max(cum, axis=0, keepdims=True)

    ntiles = jnp.floor((counts + (BR - 1)) * (1.0 / BR))
    inc = ntiles
    s = 1
    while s < E:
        inc = inc + jnp.concatenate(
            [jnp.zeros((1, s), jnp.float32), inc[:, :E - s]], axis=1)
        s *= 2
    toff = inc - ntiles

    dest = toff * BR + rank
    p1 = jnp.sum(jnp.where(lane8 == i1, dest, 0.0), axis=1, keepdims=True)
    p2 = jnp.sum(jnp.where(lane8 == i2, dest, 0.0), axis=1, keepdims=True)
    pos_ref[:, 0:1] = p1.astype(jnp.int32)
    pos_ref[:, 1:2] = p2.astype(jnp.int32)

    tile_end = toff + ntiles
    eye = jnp.where(
        jax.lax.broadcasted_iota(jnp.int32, (E, E), 0)
        == jax.lax.broadcasted_iota(jnp.int32, (E, E), 1), 1.0, 0.0)
    te_col = jnp.sum(tile_end * eye, axis=1, keepdims=True)
    ti = jax.lax.broadcasted_iota(
        jnp.int32, (E, NTILES), 1).astype(jnp.float32)
    past = jnp.where(ti >= te_col, 1.0, 0.0)
    te = jnp.sum(past, axis=0, keepdims=True)
    te_ref[0:1, :] = jnp.minimum(te, E - 1).astype(jnp.int32)
    total = jnp.max(tile_end)
    tidx = jax.lax.broadcasted_iota(
        jnp.int32, (1, NTILES), 1).astype(jnp.float32)
    te_ref[1:2, :] = jnp.where(tidx < total, 1, 0).astype(jnp.int32)

    xf = x_ref[...]
    xpk_ref[...] = _pack_bf16_pair(xf[:, :HP], xf[:, HP:])


def _router(x, gate_w):
    return pl.pallas_call(
        _router_body,
        out_shape=[
            jax.ShapeDtypeStruct((T, 2), jnp.int32),
            jax.ShapeDtypeStruct((T, 2), jnp.float32),
            jax.ShapeDtypeStruct((2, NTILES), jnp.int32),
            jax.ShapeDtypeStruct((T, HP), jnp.int32),
        ],
    )(x, gate_w)


def _dispatch(x, pos_t):
    mesh = plsc.VectorSubcoreMesh(core_axis_name="c", subcore_axis_name="s")

    @functools.partial(
        pl.kernel, mesh=mesh,
        out_type=jax.ShapeDtypeStruct((CAP, HP), jnp.int32),
        scratch_types=[
            pltpu.VMEM((TPW,), jnp.int32),
            pltpu.VMEM((TPW, HP), jnp.int32),
            pltpu.SemaphoreType.DMA,
        ],
    )
    def k(x_hbm, pos_hbm, xs_hbm, idx_v, rows_v, sem):
        wid = jax.lax.axis_index("s") * 2 + jax.lax.axis_index("c")
        base = wid * TPW
        pltpu.sync_copy(x_hbm.at[pl.ds(base, TPW), :], rows_v)
        pltpu.sync_copy(pos_hbm.at[0, pl.ds(base, TPW)], idx_v)
        pltpu.async_copy(rows_v, xs_hbm.at[idx_v], sem).wait()
        pltpu.sync_copy(pos_hbm.at[1, pl.ds(base, TPW)], idx_v)
        pltpu.async_copy(rows_v, xs_hbm.at[idx_v], sem).wait()

    return k(x, pos_t)


def _gmm_body(te_ref, xs_ref, wgu_ref, wd_ref, out_ref):
    @pl.when(te_ref[1, pl.program_id(0)] == 1)
    def _():
        xlo, xhi = _unpack_bf16_pair(xs_ref[...])
        xrow = jnp.concatenate([xlo, xhi], axis=1).astype(jnp.bfloat16)
        gu = jax.lax.dot_general(
            xrow, wgu_ref[0].astype(jnp.bfloat16),
            (((1,), (1,)), ((), ())),
            preferred_element_type=jnp.float32)
        g = gu[:, :I]
        u = gu[:, I:]
        act = (g * jax.nn.sigmoid(g) * u).astype(jnp.bfloat16)
        orow = jax.lax.dot_general(
            act, wd_ref[0].astype(jnp.bfloat16), (((1,), (1,)), ((), ())),
            preferred_element_type=jnp.float32)
        out_ref[...] = _pack_bf16_pair(orow[:, :HP], orow[:, HP:])


def _gmm(te, xs, wgu, wd):
    return pl.pallas_call(
        _gmm_body,
        grid_spec=pltpu.PrefetchScalarGridSpec(
            num_scalar_prefetch=1,
            grid=(NTILES,),
            in_specs=[
                pl.BlockSpec((BR, HP), lambda i, te: (i, 0)),
                pl.BlockSpec((1, 2 * I, H), lambda i, te: (te[0, i], 0, 0)),
                pl.BlockSpec((1, H, I), lambda i, te: (te[0, i], 0, 0)),
            ],
            out_specs=pl.BlockSpec((BR, HP), lambda i, te: (i, 0)),
        ),
        out_shape=jax.ShapeDtypeStruct((CAP, HP), jnp.int32),
        compiler_params=pltpu.CompilerParams(
            dimension_semantics=("arbitrary",)),
    )(te, xs, wgu, wd)


def _combine_gather(rows, pos_t):
    mesh = plsc.VectorSubcoreMesh(core_axis_name="c", subcore_axis_name="s")

    @functools.partial(
        pl.kernel, mesh=mesh,
        out_type=jax.ShapeDtypeStruct((2, T, HP), jnp.int32),
        scratch_types=[
            pltpu.VMEM((TPW,), jnp.int32),
            pltpu.VMEM((TPW, HP), jnp.int32),
            pltpu.SemaphoreType.DMA,
        ],
    )
    def k(rows_hbm, pos_hbm, g_hbm, idx_v, buf_v, sem):
        wid = jax.lax.axis_index("s") * 2 + jax.lax.axis_index("c")
        base = wid * TPW
        pltpu.sync_copy(pos_hbm.at[0, pl.ds(base, TPW)], idx_v)
        pltpu.async_copy(rows_hbm.at[idx_v], buf_v, sem).wait()
        pltpu.sync_copy(buf_v, g_hbm.at[0, pl.ds(base, TPW), :])
        pltpu.sync_copy(pos_hbm.at[1, pl.ds(base, TPW)], idx_v)
        pltpu.async_copy(rows_hbm.at[idx_v], buf_v, sem).wait()
        pltpu.sync_copy(buf_v, g_hbm.at[1, pl.ds(base, TPW), :])

    return k(rows, pos_t)


def _final_body(x_ref, sgu_ref, sdn_ref, g_ref, wts_ref, out_ref):
    sgu = jax.lax.dot_general(
        x_ref[...].astype(jnp.bfloat16), sgu_ref[...].astype(jnp.bfloat16),
        (((1,), (1,)), ((), ())),
        preferred_element_type=jnp.float32)
    half = sgu.shape[1] // 2
    sg = sgu[:, :half]
    su = sgu[:, half:]
    sact = (sg * jax.nn.sigmoid(sg) * su).astype(jnp.bfloat16)
    shared = jax.lax.dot_general(
        sact, sdn_ref[...].astype(jnp.bfloat16), (((1,), (1,)), ((), ())),
        preferred_element_type=jnp.float32)
    g0lo, g0hi = _unpack_bf16_pair(g_ref[0])
    g1lo, g1hi = _unpack_bf16_pair(g_ref[1])
    g0 = jnp.concatenate([g0lo, g0hi], axis=1)
    g1 = jnp.concatenate([g1lo, g1hi], axis=1)
    out_ref[...] = (shared
                    + wts_ref[:, 0:1] * g0
                    + wts_ref[:, 1:2] * g1)


def _final(x, sgu, sdn, g, wts):
    nt = T // BT
    return pl.pallas_call(
        _final_body,
        grid=(nt,),
        in_specs=[
            pl.BlockSpec((BT, H), lambda t: (t, 0)),
            pl.BlockSpec(sgu.shape, lambda t: (0, 0)),
            pl.BlockSpec(sdn.shape, lambda t: (0, 0)),
            pl.BlockSpec((2, BT, HP), lambda t: (0, t, 0)),
            pl.BlockSpec((BT, 2), lambda t: (t, 0)),
        ],
        out_specs=pl.BlockSpec((BT, H), lambda t: (t, 0)),
        out_shape=jax.ShapeDtypeStruct((T, H), jnp.float32),
        compiler_params=pltpu.CompilerParams(
            dimension_semantics=("arbitrary",)),
    )(x, sgu, sdn, g, wts)


def kernel(hidden_states, gate_w, experts_gate_up, experts_down,
           shared_gate_up, shared_down):
    x = hidden_states
    wgu = experts_gate_up
    wd = experts_down
    sgu = shared_gate_up
    sdn = shared_down

    pos, wts, te, xpk = _router(x, gate_w)
    pos_t = pos.T
    xs = _dispatch(xpk, pos_t)
    rows = _gmm(te, xs, wgu, wd)
    g = _combine_gather(rows, pos_t)
    return _final(x, sgu, sdn, g, wts)


# speedup vs baseline: 1.1833x; 1.1833x over previous
import functools

import jax
import jax.numpy as jnp
from jax.experimental import pallas as pl
from jax.experimental.pallas import tpu as pltpu
from jax.experimental.pallas import tpu_sc as plsc

T = 2048
H = 1024
I = 512
E = 8
ROUTED_SCALE = 2.5

BR = 256
NTILES = T * 2 // BR + E
CAP = NTILES * BR
BT = 256

NW = 32
TPW = T // NW


HP = H // 2


def _pack_bf16_pair(lo_f32, hi_f32):
    def bits(v):
        r = jax.lax.bitcast_convert_type(v, jnp.int32)
        lsb = jax.lax.shift_right_logical(r, 16) & 1
        return jax.lax.shift_right_logical(r + 0x7FFF + lsb, 16)
    return jax.lax.shift_left(bits(hi_f32), 16) | bits(lo_f32)


def _unpack_bf16_pair(pk):
    lo = jax.lax.bitcast_convert_type(jax.lax.shift_left(pk, 16), jnp.float32)
    hi = jax.lax.bitcast_convert_type(
        jax.lax.shift_left(jax.lax.shift_right_arithmetic(pk, 16), 16),
        jnp.float32)
    return lo, hi


def _router_body(x_ref, gate_ref, pos_ref, wts_ref, te_ref, xpk_ref):
    lane8 = jax.lax.broadcasted_iota(jnp.int32, (T, E), 1)
    xb = x_ref[...].astype(jnp.bfloat16)
    logits = jax.lax.dot_general(
        xb, gate_ref[...].astype(jnp.bfloat16), (((1,), (1,)), ((), ())),
        preferred_element_type=jnp.float32)
    m = jnp.max(logits, axis=1, keepdims=True)
    ex = jnp.exp(logits - m)
    scores = ex / jnp.sum(ex, axis=1, keepdims=True)
    neg = jnp.float32(-1e30)
    m0 = jnp.max(jnp.where(lane8 < 4, scores, neg), axis=1, keepdims=True)
    m1 = jnp.max(jnp.where(lane8 >= 4, scores, neg), axis=1, keepdims=True)
    low_f = jnp.where(lane8 < 4, 1.0, 0.0)
    pick_low = jnp.where(m0 >= m1, 1.0, 0.0)
    gmask = pick_low * low_f + (1.0 - pick_low) * (1.0 - low_f)
    ms = scores * gmask
    v1 = jnp.max(ms, axis=1, keepdims=True)
    i1 = jnp.min(jnp.where(ms == v1, lane8, E), axis=1, keepdims=True)
    ms2 = jnp.where(lane8 == i1, -1.0, ms)
    v2 = jnp.max(ms2, axis=1, keepdims=True)
    i2 = jnp.min(jnp.where(ms2 == v2, lane8, E), axis=1, keepdims=True)
    denom = v1 + v2 + 1e-20
    wts_ref[:, 0:1] = v1 / denom * ROUTED_SCALE
    wts_ref[:, 1:2] = v2 / denom * ROUTED_SCALE

    oh = (jnp.where(lane8 == i1, 1.0, 0.0)
          + jnp.where(lane8 == i2, 1.0, 0.0))
    cum = oh
    s = 1
    while s < T:
        cum = cum + jnp.concatenate(
            [jnp.zeros((s, E), jnp.float32), cum[:T - s, :]], axis=0)
        s *= 2
    rank = cum - oh
    counts = jnp.max(cum, axis=0, keepdims=True)

    ntiles = jnp.floor((counts + (BR - 1)) * (1.0 / BR))
    inc = ntiles
    s = 1
    while s < E:
        inc = inc + jnp.concatenate(
            [jnp.zeros((1, s), jnp.float32), inc[:, :E - s]], axis=1)
        s *= 2
    toff = inc - ntiles

    dest = toff * BR + rank
    p1 = jnp.sum(jnp.where(lane8 == i1, dest, 0.0), axis=1, keepdims=True)
    p2 = jnp.sum(jnp.where(lane8 == i2, dest, 0.0), axis=1, keepdims=True)
    pos_ref[:, 0:1] = p1.astype(jnp.int32)
    pos_ref[:, 1:2] = p2.astype(jnp.int32)

    tile_end = toff + ntiles
    eye = jnp.where(
        jax.lax.broadcasted_iota(jnp.int32, (E, E), 0)
        == jax.lax.broadcasted_iota(jnp.int32, (E, E), 1), 1.0, 0.0)
    te_col = jnp.sum(tile_end * eye, axis=1, keepdims=True)
    ti = jax.lax.broadcasted_iota(
        jnp.int32, (E, NTILES), 1).astype(jnp.float32)
    past = jnp.where(ti >= te_col, 1.0, 0.0)
    te = jnp.sum(past, axis=0, keepdims=True)
    te_ref[0:1, :] = jnp.minimum(te, E - 1).astype(jnp.int32)
    total = jnp.max(tile_end)
    tidx = jax.lax.broadcasted_iota(
        jnp.int32, (1, NTILES), 1).astype(jnp.float32)
    te_ref[1:2, :] = jnp.where(tidx < total, 1, 0).astype(jnp.int32)

    xf = x_ref[...]
    xpk_ref[...] = _pack_bf16_pair(xf[:, :HP], xf[:, HP:])


def _router(x, gate_w):
    return pl.pallas_call(
        _router_body,
        out_shape=[
            jax.ShapeDtypeStruct((T, 2), jnp.int32),
            jax.ShapeDtypeStruct((T, 2), jnp.float32),
            jax.ShapeDtypeStruct((2, NTILES), jnp.int32),
            jax.ShapeDtypeStruct((T, HP), jnp.int32),
        ],
    )(x, gate_w)


def _dispatch(x, pos_t):
    mesh = plsc.VectorSubcoreMesh(core_axis_name="c", subcore_axis_name="s")

    @functools.partial(
        pl.kernel, mesh=mesh,
        out_type=jax.ShapeDtypeStruct((CAP, HP), jnp.int32),
        scratch_types=[
            pltpu.VMEM((TPW,), jnp.int32),
            pltpu.VMEM((TPW, HP), jnp.int32),
            pltpu.SemaphoreType.DMA,
        ],
    )
    def k(x_hbm, pos_hbm, xs_hbm, idx_v, rows_v, sem):
        wid = jax.lax.axis_index("s") * 2 + jax.lax.axis_index("c")
        base = wid * TPW
        pltpu.sync_copy(x_hbm.at[pl.ds(base, TPW), :], rows_v)
        pltpu.sync_copy(pos_hbm.at[0, pl.ds(base, TPW)], idx_v)
        pltpu.async_copy(rows_v, xs_hbm.at[idx_v], sem).wait()
        pltpu.sync_copy(pos_hbm.at[1, pl.ds(base, TPW)], idx_v)
        pltpu.async_copy(rows_v, xs_hbm.at[idx_v], sem).wait()

    return k(x, pos_t)


def _gmm_body(te_ref, xs_ref, wgu_ref, wd_ref, out_ref):
    @pl.when(te_ref[1, pl.program_id(0)] == 1)
    def _():
        xlo, xhi = _unpack_bf16_pair(xs_ref[...])
        xrow = jnp.concatenate([xlo, xhi], axis=1).astype(jnp.bfloat16)
        gu = jax.lax.dot_general(
            xrow, wgu_ref[0].astype(jnp.bfloat16),
            (((1,), (1,)), ((), ())),
            preferred_element_type=jnp.float32)
        g = gu[:, :I]
        u = gu[:, I:]
        act = (g * jax.nn.sigmoid(g) * u).astype(jnp.bfloat16)
        orow = jax.lax.dot_general(
            act, wd_ref[0].astype(jnp.bfloat16), (((1,), (1,)), ((), ())),
            preferred_element_type=jnp.float32)
        out_ref[...] = _pack_bf16_pair(orow[:, :HP], orow[:, HP:])


def _gmm(te, xs, wgu, wd):
    return pl.pallas_call(
        _gmm_body,
        grid_spec=pltpu.PrefetchScalarGridSpec(
            num_scalar_prefetch=1,
            grid=(NTILES,),
            in_specs=[
                pl.BlockSpec((BR, HP), lambda i, te: (i, 0)),
                pl.BlockSpec((1, 2 * I, H), lambda i, te: (te[0, i], 0, 0)),
                pl.BlockSpec((1, H, I), lambda i, te: (te[0, i], 0, 0)),
            ],
            out_specs=pl.BlockSpec((BR, HP), lambda i, te: (i, 0)),
        ),
        out_shape=jax.ShapeDtypeStruct((CAP, HP), jnp.int32),
        compiler_params=pltpu.CompilerParams(
            dimension_semantics=("arbitrary",)),
    )(te, xs, wgu, wd)


def _combine_gather(rows, pos_t):
    mesh = plsc.VectorSubcoreMesh(core_axis_name="c", subcore_axis_name="s")

    @functools.partial(
        pl.kernel, mesh=mesh,
        out_type=jax.ShapeDtypeStruct((2, T, HP), jnp.int32),
        scratch_types=[
            pltpu.VMEM((TPW,), jnp.int32),
            pltpu.VMEM((TPW, HP), jnp.int32),
            pltpu.SemaphoreType.DMA,
        ],
    )
    def k(rows_hbm, pos_hbm, g_hbm, idx_v, buf_v, sem):
        wid = jax.lax.axis_index("s") * 2 + jax.lax.axis_index("c")
        base = wid * TPW
        pltpu.sync_copy(pos_hbm.at[0, pl.ds(base, TPW)], idx_v)
        pltpu.async_copy(rows_hbm.at[idx_v], buf_v, sem).wait()
        pltpu.sync_copy(buf_v, g_hbm.at[0, pl.ds(base, TPW), :])
        pltpu.sync_copy(pos_hbm.at[1, pl.ds(base, TPW)], idx_v)
        pltpu.async_copy(rows_hbm.at[idx_v], buf_v, sem).wait()
        pltpu.sync_copy(buf_v, g_hbm.at[1, pl.ds(base, TPW), :])

    return k(rows, pos_t)


def _final_body(x_ref, sgu_ref, sdn_ref, g_ref, wts_ref, out_ref):
    sgu = jax.lax.dot_general(
        x_ref[...].astype(jnp.bfloat16), sgu_ref[...].astype(jnp.bfloat16),
        (((1,), (1,)), ((), ())),
        preferred_element_type=jnp.float32)
    half = sgu.shape[1] // 2
    sg = sgu[:, :half]
    su = sgu[:, half:]
    sact = (sg * jax.nn.sigmoid(sg) * su).astype(jnp.bfloat16)
    shared = jax.lax.dot_general(
        sact, sdn_ref[...].astype(jnp.bfloat16), (((1,), (1,)), ((), ())),
        preferred_element_type=jnp.float32)
    g0lo, g0hi = _unpack_bf16_pair(g_ref[0])
    g1lo, g1hi = _unpack_bf16_pair(g_ref[1])
    g0 = jnp.concatenate([g0lo, g0hi], axis=1)
    g1 = jnp.concatenate([g1lo, g1hi], axis=1)
    out_ref[...] = (shared
                    + wts_ref[:, 0:1] * g0
                    + wts_ref[:, 1:2] * g1)


def _final(x, sgu, sdn, g, wts):
    nt = T // BT
    return pl.pallas_call(
        _final_body,
        grid=(nt,),
        in_specs=[
            pl.BlockSpec((BT, H), lambda t: (t, 0)),
            pl.BlockSpec(sgu.shape, lambda t: (0, 0)),
            pl.BlockSpec(sdn.shape, lambda t: (0, 0)),
            pl.BlockSpec((2, BT, HP), lambda t: (0, t, 0)),
            pl.BlockSpec((BT, 2), lambda t: (t, 0)),
        ],
        out_specs=pl.BlockSpec((BT, H), lambda t: (t, 0)),
        out_shape=jax.ShapeDtypeStruct((T, H), jnp.float32),
        compiler_params=pltpu.CompilerParams(
            dimension_semantics=("arbitrary",)),
    )(x, sgu, sdn, g, wts)


def kernel(hidden_states, gate_w, experts_gate_up, experts_down,
           shared_gate_up, shared_down):
    x = hidden_states
    wgu = experts_gate_up
    wd = experts_down
    sgu = shared_gate_up
    sdn = shared_down

    pos, wts, te, xpk = _router(x, gate_w)
    pos_t = pos.T
    xs = _dispatch(xpk, pos_t)
    rows = _gmm(te, xs, wgu, wd)
    g = _combine_gather(rows, pos_t)
    return _final(x, sgu, sdn, g, wts)
